# baseline (speedup 1.0000x reference)
import jax
import jax.numpy as jnp
from jax.experimental import pallas as pl
from jax.experimental.pallas import tpu as pltpu

_B, _IN_DIM, _HID, _LAT, _E_DIM, _NQ, _NE = 16384, 768, 512, 256, 64, 4, 256
_BETA = 0.25
_TB = 1024

_PREC = jax.lax.Precision.DEFAULT


def _fused_body(x_ref, w1_ref, b1_ref, w2_ref, b2_ref, wd_ref,
                cbbd_ref, csq_ref, dw1_ref, db1_ref, dw2_ref, db2_ref,
                out_ref, i0_ref, i1_ref, i2_ref, i3_ref, loss_ref):
    step = pl.program_id(0)

    h = jnp.maximum(
        jnp.dot(x_ref[...], w1_ref[...], precision=_PREC) + b1_ref[...], 0.0)
    z = jnp.dot(h, w2_ref[...], precision=_PREC) + b2_ref[...]

    mm_all = jnp.dot(z, wd_ref[...], precision=_PREC)

    idx_refs = (i0_ref, i1_ref, i2_ref, i3_ref)
    iota_f = jax.lax.broadcasted_iota(
        jnp.int32, (_TB, _NE), 1).astype(jnp.float32)
    oh_parts = []
    for q in range(_NQ):
        zh = z[:, q * _E_DIM:(q + 1) * _E_DIM]
        zsq = jnp.sum(zh * zh, axis=1, keepdims=True)
        d = (zsq + csq_ref[q:q + 1, :]) \
            - mm_all[:, q * _NE:(q + 1) * _NE]
        dmin = jnp.min(d, axis=-1, keepdims=True)
        hit = d == dmin
        idxf = jnp.min(jnp.where(hit, iota_f, float(_NE)), axis=-1,
                       keepdims=True)
        idx_refs[q][...] = idxf.astype(jnp.int32)
        oh_parts.append((iota_f == idxf).astype(jnp.float32))
    onehot_all = jnp.concatenate(oh_parts, axis=1)

    zq_all = jnp.dot(onehot_all, cbbd_ref[...], precision=_PREC)
    diff = zq_all - z
    x_q = z + diff

    h2 = jnp.maximum(
        jnp.dot(x_q, dw1_ref[...], precision=_PREC) + db1_ref[...], 0.0)
    out_ref[...] = jnp.dot(h2, dw2_ref[...], precision=_PREC) + db2_ref[...]

    @pl.when(step == 0)
    def _():
        loss_ref[0, 0] = 0.0
    loss_ref[0, 0] += jnp.sum(diff * diff)


def kernel(x, enc_w1, enc_b1, enc_w2, enc_b2, codebooks,
           dec_w1, dec_b1, dec_w2, dec_b2):
    eye = jnp.eye(_NQ, dtype=jnp.float32)
    cbT = jnp.swapaxes(codebooks, 1, 2)
    wd = (2.0 * eye)[:, None, :, None] * cbT[:, :, None, :]
    wd = wd.reshape(_NQ * _E_DIM, _NQ * _NE)
    cbbd = (eye[:, None, :, None] * codebooks[:, :, None, :]).reshape(
        _NQ * _NE, _NQ * _E_DIM)
    csq = jnp.sum(codebooks * codebooks, axis=2)

    grid = (_B // _TB,)
    const = lambda *shape: pl.BlockSpec(shape, lambda i: (0,) * len(shape))
    out, i0, i1, i2, i3, loss_sum = pl.pallas_call(
        _fused_body,
        grid=grid,
        in_specs=[
            pl.BlockSpec((_TB, _IN_DIM), lambda i: (i, 0)),
            const(_IN_DIM, _HID),
            const(1, _HID),
            const(_HID, _LAT),
            const(1, _LAT),
            const(_NQ * _E_DIM, _NQ * _NE),
            const(_NQ * _NE, _NQ * _E_DIM),
            const(_NQ, _NE),
            const(_LAT, _HID),
            const(1, _HID),
            const(_HID, _IN_DIM),
            const(1, _IN_DIM),
        ],
        out_specs=[
            pl.BlockSpec((_TB, _IN_DIM), lambda i: (i, 0)),
            pl.BlockSpec((_TB, 1), lambda i: (i, 0)),
            pl.BlockSpec((_TB, 1), lambda i: (i, 0)),
            pl.BlockSpec((_TB, 1), lambda i: (i, 0)),
            pl.BlockSpec((_TB, 1), lambda i: (i, 0)),
            pl.BlockSpec(memory_space=pltpu.SMEM, block_shape=(1, 1),
                         index_map=lambda i: (0, 0)),
        ],
        out_shape=[
            jax.ShapeDtypeStruct((_B, _IN_DIM), jnp.float32),
            jax.ShapeDtypeStruct((_B, 1), jnp.int32),
            jax.ShapeDtypeStruct((_B, 1), jnp.int32),
            jax.ShapeDtypeStruct((_B, 1), jnp.int32),
            jax.ShapeDtypeStruct((_B, 1), jnp.int32),
            jax.ShapeDtypeStruct((1, 1), jnp.float32),
        ],
        compiler_params=pltpu.CompilerParams(
            dimension_semantics=("arbitrary",)),
    )(x, enc_w1, enc_b1.reshape(1, _HID), enc_w2, enc_b2.reshape(1, _LAT),
      wd, cbbd, csq, dec_w1, dec_b1.reshape(1, _HID), dec_w2,
      dec_b2.reshape(1, _IN_DIM))

    quant_loss = loss_sum[0, 0] * ((1.0 + _BETA) / (_B * _LAT))
    indices = jnp.concatenate([i0, i1, i2, i3], axis=1)
    return out, quant_loss, indices


# speedup vs baseline: 1.1192x; 1.1192x over previous
import jax
import jax.numpy as jnp
from jax.experimental import pallas as pl
from jax.experimental.pallas import tpu as pltpu

_B, _IN_DIM, _HID, _LAT, _E_DIM, _NQ, _NE = 16384, 768, 512, 256, 64, 4, 256
_BETA = 0.25
_TB = 1024

_PREC = jax.lax.Precision.DEFAULT


def _fused_body(x_ref, w1_ref, b1_ref, w2_ref, b2_ref, wd_ref,
                cbbd_ref, csq_ref, dw1_ref, db1_ref, dw2_ref, db2_ref,
                out_ref, i0_ref, i1_ref, i2_ref, i3_ref, loss_ref):
    step = pl.program_id(0)

    h = jnp.maximum(
        jnp.dot(x_ref[...], w1_ref[...], precision=_PREC) + b1_ref[...], 0.0)
    z = jnp.dot(h, w2_ref[...], precision=_PREC) + b2_ref[...]

    mm_all = jnp.dot(z, wd_ref[...], precision=_PREC)

    idx_refs = (i0_ref, i1_ref, i2_ref, i3_ref)
    iota_f = jax.lax.broadcasted_iota(
        jnp.int32, (_TB, _NE), 1).astype(jnp.float32)
    oh_parts = []
    for q in range(_NQ):
        zh = z[:, q * _E_DIM:(q + 1) * _E_DIM]
        zsq = jnp.sum(zh * zh, axis=1, keepdims=True)
        d = (zsq + csq_ref[q:q + 1, :]) \
            - mm_all[:, q * _NE:(q + 1) * _NE]
        dmin = jnp.min(d, axis=-1, keepdims=True)
        hit = d == dmin
        idxf = jnp.min(jnp.where(hit, iota_f, float(_NE)), axis=-1,
                       keepdims=True)
        idx_refs[q][...] = idxf[:, 0].astype(jnp.int32)
        oh_parts.append((iota_f == idxf).astype(jnp.float32))
    onehot_all = jnp.concatenate(oh_parts, axis=1)

    zq_all = jnp.dot(onehot_all, cbbd_ref[...], precision=_PREC)
    diff = zq_all - z
    x_q = z + diff

    h2 = jnp.maximum(
        jnp.dot(x_q, dw1_ref[...], precision=_PREC) + db1_ref[...], 0.0)
    out_ref[...] = jnp.dot(h2, dw2_ref[...], precision=_PREC) + db2_ref[...]

    @pl.when(step == 0)
    def _():
        loss_ref[0, 0] = 0.0
    loss_ref[0, 0] += jnp.sum(diff * diff)


def kernel(x, enc_w1, enc_b1, enc_w2, enc_b2, codebooks,
           dec_w1, dec_b1, dec_w2, dec_b2):
    eye = jnp.eye(_NQ, dtype=jnp.float32)
    cbT = jnp.swapaxes(codebooks, 1, 2)
    wd = (2.0 * eye)[:, None, :, None] * cbT[:, :, None, :]
    wd = wd.reshape(_NQ * _E_DIM, _NQ * _NE)
    cbbd = (eye[:, None, :, None] * codebooks[:, :, None, :]).reshape(
        _NQ * _NE, _NQ * _E_DIM)
    csq = jnp.sum(codebooks * codebooks, axis=2)

    grid = (_B // _TB,)
    const = lambda *shape: pl.BlockSpec(shape, lambda i: (0,) * len(shape))
    out, i0, i1, i2, i3, loss_sum = pl.pallas_call(
        _fused_body,
        grid=grid,
        in_specs=[
            pl.BlockSpec((_TB, _IN_DIM), lambda i: (i, 0)),
            const(_IN_DIM, _HID),
            const(1, _HID),
            const(_HID, _LAT),
            const(1, _LAT),
            const(_NQ * _E_DIM, _NQ * _NE),
            const(_NQ * _NE, _NQ * _E_DIM),
            const(_NQ, _NE),
            const(_LAT, _HID),
            const(1, _HID),
            const(_HID, _IN_DIM),
            const(1, _IN_DIM),
        ],
        out_specs=[
            pl.BlockSpec((_TB, _IN_DIM), lambda i: (i, 0)),
            pl.BlockSpec((_TB,), lambda i: (i,)),
            pl.BlockSpec((_TB,), lambda i: (i,)),
            pl.BlockSpec((_TB,), lambda i: (i,)),
            pl.BlockSpec((_TB,), lambda i: (i,)),
            pl.BlockSpec(memory_space=pltpu.SMEM, block_shape=(1, 1),
                         index_map=lambda i: (0, 0)),
        ],
        out_shape=[
            jax.ShapeDtypeStruct((_B, _IN_DIM), jnp.float32),
            jax.ShapeDtypeStruct((_B,), jnp.int32),
            jax.ShapeDtypeStruct((_B,), jnp.int32),
            jax.ShapeDtypeStruct((_B,), jnp.int32),
            jax.ShapeDtypeStruct((_B,), jnp.int32),
            jax.ShapeDtypeStruct((1, 1), jnp.float32),
        ],
        compiler_params=pltpu.CompilerParams(
            dimension_semantics=("arbitrary",)),
    )(x, enc_w1, enc_b1.reshape(1, _HID), enc_w2, enc_b2.reshape(1, _LAT),
      wd, cbbd, csq, dec_w1, dec_b1.reshape(1, _HID), dec_w2,
      dec_b2.reshape(1, _IN_DIM))

    quant_loss = loss_sum[0, 0] * ((1.0 + _BETA) / (_B * _LAT))
    indices = jnp.stack([i0, i1, i2, i3], axis=-1)
    return out, quant_loss, indices


# speedup vs baseline: 1.1601x; 1.0366x over previous
import jax
import jax.numpy as jnp
from jax.experimental import pallas as pl
from jax.experimental.pallas import tpu as pltpu

_B, _IN_DIM, _HID, _LAT, _E_DIM, _NQ, _NE = 16384, 768, 512, 256, 64, 4, 256
_BETA = 0.25
_TB = 2048

_PREC = jax.lax.Precision.DEFAULT


def _fused_body(x_ref, w1_ref, b1_ref, w2_ref, b2_ref, wd_ref,
                cbbd_ref, csq_ref, dw1_ref, db1_ref, dw2_ref, db2_ref,
                out_ref, i0_ref, i1_ref, i2_ref, i3_ref, loss_ref):
    step = pl.program_id(0)

    h = jnp.maximum(
        jnp.dot(x_ref[...], w1_ref[...], precision=_PREC) + b1_ref[...], 0.0)
    z = jnp.dot(h, w2_ref[...], precision=_PREC) + b2_ref[...]

    mm_all = jnp.dot(z, wd_ref[...], precision=_PREC)

    idx_refs = (i0_ref, i1_ref, i2_ref, i3_ref)
    iota_f = jax.lax.broadcasted_iota(
        jnp.int32, (_TB, _NE), 1).astype(jnp.float32)
    oh_parts = []
    for q in range(_NQ):
        zh = z[:, q * _E_DIM:(q + 1) * _E_DIM]
        zsq = jnp.sum(zh * zh, axis=1, keepdims=True)
        d = (zsq + csq_ref[q:q + 1, :]) \
            - mm_all[:, q * _NE:(q + 1) * _NE]
        dmin = jnp.min(d, axis=-1, keepdims=True)
        hit = d == dmin
        idxf = jnp.min(jnp.where(hit, iota_f, float(_NE)), axis=-1,
                       keepdims=True)
        idx_refs[q][...] = idxf[:, 0].astype(jnp.int32)
        oh_parts.append((iota_f == idxf).astype(jnp.float32))
    onehot_all = jnp.concatenate(oh_parts, axis=1)

    zq_all = jnp.dot(onehot_all, cbbd_ref[...], precision=_PREC)
    diff = zq_all - z
    x_q = z + diff

    h2 = jnp.maximum(
        jnp.dot(x_q, dw1_ref[...], precision=_PREC) + db1_ref[...], 0.0)
    out_ref[...] = jnp.dot(h2, dw2_ref[...], precision=_PREC) + db2_ref[...]

    @pl.when(step == 0)
    def _():
        loss_ref[0, 0] = 0.0
    loss_ref[0, 0] += jnp.sum(diff * diff)


def kernel(x, enc_w1, enc_b1, enc_w2, enc_b2, codebooks,
           dec_w1, dec_b1, dec_w2, dec_b2):
    eye = jnp.eye(_NQ, dtype=jnp.float32)
    cbT = jnp.swapaxes(codebooks, 1, 2)
    wd = (2.0 * eye)[:, None, :, None] * cbT[:, :, None, :]
    wd = wd.reshape(_NQ * _E_DIM, _NQ * _NE)
    cbbd = (eye[:, None, :, None] * codebooks[:, :, None, :]).reshape(
        _NQ * _NE, _NQ * _E_DIM)
    csq = jnp.sum(codebooks * codebooks, axis=2)

    grid = (_B // _TB,)
    const = lambda *shape: pl.BlockSpec(shape, lambda i: (0,) * len(shape))
    out, i0, i1, i2, i3, loss_sum = pl.pallas_call(
        _fused_body,
        grid=grid,
        in_specs=[
            pl.BlockSpec((_TB, _IN_DIM), lambda i: (i, 0)),
            const(_IN_DIM, _HID),
            const(1, _HID),
            const(_HID, _LAT),
            const(1, _LAT),
            const(_NQ * _E_DIM, _NQ * _NE),
            const(_NQ * _NE, _NQ * _E_DIM),
            const(_NQ, _NE),
            const(_LAT, _HID),
            const(1, _HID),
            const(_HID, _IN_DIM),
            const(1, _IN_DIM),
        ],
        out_specs=[
            pl.BlockSpec((_TB, _IN_DIM), lambda i: (i, 0)),
            pl.BlockSpec((_TB,), lambda i: (i,)),
            pl.BlockSpec((_TB,), lambda i: (i,)),
            pl.BlockSpec((_TB,), lambda i: (i,)),
            pl.BlockSpec((_TB,), lambda i: (i,)),
            pl.BlockSpec(memory_space=pltpu.SMEM, block_shape=(1, 1),
                         index_map=lambda i: (0, 0)),
        ],
        out_shape=[
            jax.ShapeDtypeStruct((_B, _IN_DIM), jnp.float32),
            jax.ShapeDtypeStruct((_B,), jnp.int32),
            jax.ShapeDtypeStruct((_B,), jnp.int32),
            jax.ShapeDtypeStruct((_B,), jnp.int32),
            jax.ShapeDtypeStruct((_B,), jnp.int32),
            jax.ShapeDtypeStruct((1, 1), jnp.float32),
        ],
        compiler_params=pltpu.CompilerParams(
            dimension_semantics=("arbitrary",)),
    )(x, enc_w1, enc_b1.reshape(1, _HID), enc_w2, enc_b2.reshape(1, _LAT),
      wd, cbbd, csq, dec_w1, dec_b1.reshape(1, _HID), dec_w2,
      dec_b2.reshape(1, _IN_DIM))

    quant_loss = loss_sum[0, 0] * ((1.0 + _BETA) / (_B * _LAT))
    indices = jnp.stack([i0, i1, i2, i3], axis=-1)
    return out, quant_loss, indices


# speedup vs baseline: 1.1617x; 1.0013x over previous
import jax
import jax.numpy as jnp
from jax.experimental import pallas as pl
from jax.experimental.pallas import tpu as pltpu

_B, _IN_DIM, _HID, _LAT, _E_DIM, _NQ, _NE = 16384, 768, 512, 256, 64, 4, 256
_BETA = 0.25
_TB = 2048

_PREC = jax.lax.Precision.DEFAULT


_NSUB = 2
_HB = _TB // _NSUB


def _fused_body(x_ref, w1_ref, b1_ref, w2_ref, b2_ref, wd_ref,
                cbbd_ref, csq_ref, dw1_ref, db1_ref, dw2_ref, db2_ref,
                out_ref, i0_ref, i1_ref, i2_ref, i3_ref, loss_ref):
    step = pl.program_id(0)

    idx_refs = (i0_ref, i1_ref, i2_ref, i3_ref)
    iota_f = jax.lax.broadcasted_iota(
        jnp.int32, (_HB, _NE), 1).astype(jnp.float32)
    loss_parts = []
    for p in range(_NSUB):
        rows = pl.ds(p * _HB, _HB)
        h = jnp.maximum(
            jnp.dot(x_ref[rows, :], w1_ref[...], precision=_PREC)
            + b1_ref[...], 0.0)
        z = jnp.dot(h, w2_ref[...], precision=_PREC) + b2_ref[...]

        mm_all = jnp.dot(z, wd_ref[...], precision=_PREC)

        oh_parts = []
        for q in range(_NQ):
            zh = z[:, q * _E_DIM:(q + 1) * _E_DIM]
            zsq = jnp.sum(zh * zh, axis=1, keepdims=True)
            d = (zsq + csq_ref[q:q + 1, :]) \
                - mm_all[:, q * _NE:(q + 1) * _NE]
            dmin = jnp.min(d, axis=-1, keepdims=True)
            hit = d == dmin
            idxf = jnp.min(jnp.where(hit, iota_f, float(_NE)), axis=-1,
                           keepdims=True)
            idx_refs[q][rows] = idxf[:, 0].astype(jnp.int32)
            oh_parts.append((iota_f == idxf).astype(jnp.float32))
        onehot_all = jnp.concatenate(oh_parts, axis=1)

        zq_all = jnp.dot(onehot_all, cbbd_ref[...], precision=_PREC)
        diff = zq_all - z
        x_q = z + diff

        h2 = jnp.maximum(
            jnp.dot(x_q, dw1_ref[...], precision=_PREC) + db1_ref[...], 0.0)
        out_ref[rows, :] = jnp.dot(h2, dw2_ref[...], precision=_PREC) \
            + db2_ref[...]
        loss_parts.append(jnp.sum(diff * diff))

    @pl.when(step == 0)
    def _():
        loss_ref[0, 0] = 0.0
    loss_ref[0, 0] += sum(loss_parts)


def kernel(x, enc_w1, enc_b1, enc_w2, enc_b2, codebooks,
           dec_w1, dec_b1, dec_w2, dec_b2):
    eye = jnp.eye(_NQ, dtype=jnp.float32)
    cbT = jnp.swapaxes(codebooks, 1, 2)
    wd = (2.0 * eye)[:, None, :, None] * cbT[:, :, None, :]
    wd = wd.reshape(_NQ * _E_DIM, _NQ * _NE)
    cbbd = (eye[:, None, :, None] * codebooks[:, :, None, :]).reshape(
        _NQ * _NE, _NQ * _E_DIM)
    csq = jnp.sum(codebooks * codebooks, axis=2)

    grid = (_B // _TB,)
    const = lambda *shape: pl.BlockSpec(shape, lambda i: (0,) * len(shape))
    out, i0, i1, i2, i3, loss_sum = pl.pallas_call(
        _fused_body,
        grid=grid,
        in_specs=[
            pl.BlockSpec((_TB, _IN_DIM), lambda i: (i, 0)),
            const(_IN_DIM, _HID),
            const(1, _HID),
            const(_HID, _LAT),
            const(1, _LAT),
            const(_NQ * _E_DIM, _NQ * _NE),
            const(_NQ * _NE, _NQ * _E_DIM),
            const(_NQ, _NE),
            const(_LAT, _HID),
            const(1, _HID),
            const(_HID, _IN_DIM),
            const(1, _IN_DIM),
        ],
        out_specs=[
            pl.BlockSpec((_TB, _IN_DIM), lambda i: (i, 0)),
            pl.BlockSpec((_TB,), lambda i: (i,)),
            pl.BlockSpec((_TB,), lambda i: (i,)),
            pl.BlockSpec((_TB,), lambda i: (i,)),
            pl.BlockSpec((_TB,), lambda i: (i,)),
            pl.BlockSpec(memory_space=pltpu.SMEM, block_shape=(1, 1),
                         index_map=lambda i: (0, 0)),
        ],
        out_shape=[
            jax.ShapeDtypeStruct((_B, _IN_DIM), jnp.float32),
            jax.ShapeDtypeStruct((_B,), jnp.int32),
            jax.ShapeDtypeStruct((_B,), jnp.int32),
            jax.ShapeDtypeStruct((_B,), jnp.int32),
            jax.ShapeDtypeStruct((_B,), jnp.int32),
            jax.ShapeDtypeStruct((1, 1), jnp.float32),
        ],
        compiler_params=pltpu.CompilerParams(
            dimension_semantics=("arbitrary",)),
    )(x, enc_w1, enc_b1.reshape(1, _HID), enc_w2, enc_b2.reshape(1, _LAT),
      wd, cbbd, csq, dec_w1, dec_b1.reshape(1, _HID), dec_w2,
      dec_b2.reshape(1, _IN_DIM))

    quant_loss = loss_sum[0, 0] * ((1.0 + _BETA) / (_B * _LAT))
    indices = jnp.stack([i0, i1, i2, i3], axis=-1)
    return out, quant_loss, indices
